# Initial kernel scaffold; baseline (speedup 1.0000x reference)
import jax
import jax.numpy as jnp
from jax.experimental import pallas as pl


def kernel(seq, x, pos_enc, batch_idx, c_idx, fc_in_w1, fc_in_b1, fc_in_w2, fc_in_b2, fcs_w1, fcs_b1, fcs_w2, fcs_b2, proj_w, proj_b, qg_w, qg_b, kg_w, kg_b, vg_w, vg_b, vq_k, vq_v, ff_w1, ff_b1, ff_w2, ff_b2, out_w, out_b):
    raise NotImplementedError("write your pallas kernel here")


# speedup vs baseline: 1.3600x; 1.3600x over previous
import functools
import math

import jax
import jax.numpy as jnp
from jax import lax
from jax.experimental import pallas as pl
from jax.experimental.pallas import tpu as pltpu
from jax.experimental.pallas import tpu_sc as plsc


@functools.lru_cache(maxsize=None)
def _make_sc_hist(NN: int, C: int):
    L = 16
    NW = 32
    CH = (NN // NW) // L * L
    assert CH % 8 == 0
    TAIL = NN - CH * NW
    assert TAIL % L == 0 and (CH * NW) % 8 == 0

    mesh = plsc.VectorSubcoreMesh(core_axis_name="c", subcore_axis_name="s")

    @functools.partial(
        pl.kernel,
        mesh=mesh,
        out_type=jax.ShapeDtypeStruct((NW, C), jnp.int32),
        scratch_types=[
            pltpu.VMEM((CH,), jnp.int32),
            pltpu.VMEM((L * C,), jnp.int32),
            pltpu.VMEM((C,), jnp.int32),
            pltpu.VMEM((max(TAIL, L),), jnp.int32),
        ],
        compiler_params=pltpu.CompilerParams(needs_layout_passes=False),
    )
    def sc_hist(idx_hbm, out_hbm, idx_v, hist_v, part_v, tail_v):
        wid = lax.axis_index("s") * 2 + lax.axis_index("c")
        base = wid * CH
        pltpu.sync_copy(idx_hbm.at[pl.ds(base, CH)], idx_v)

        lane_base = lax.iota(jnp.int32, 16) * C
        ones = jnp.ones((L,), jnp.int32)
        zeros = jnp.zeros((L,), jnp.int32)

        def zbody(j, _):
            hist_v[pl.ds(j * L, L)] = zeros
            return _
        lax.fori_loop(0, (L * C) // L, zbody, 0, unroll=8)

        def sbody(i, _):
            vals = idx_v[pl.ds(i * L, L)]
            plsc.addupdate_scatter(hist_v, [lane_base + vals], ones)
            return _
        lax.fori_loop(0, CH // L, sbody, 0, unroll=8)

        if TAIL > 0:
            @pl.when(wid == 0)
            def _tail():
                pltpu.sync_copy(idx_hbm.at[pl.ds(CH * NW, TAIL)], tail_v)
                def tbody(i, _):
                    vals = tail_v[pl.ds(i * L, L)]
                    plsc.addupdate_scatter(hist_v, [lane_base + vals], ones)
                    return _
                lax.fori_loop(0, TAIL // L, tbody, 0, unroll=4)

        def rbody(j, _):
            acc = hist_v[pl.ds(j * L, L)]
            for l in range(1, L):
                acc = acc + hist_v[pl.ds(j * L + l * C, L)]
            part_v[pl.ds(j * L, L)] = acc
            return _
        lax.fori_loop(0, C // L, rbody, 0, unroll=4)

        pltpu.sync_copy(part_v, out_hbm.at[wid])

    return sc_hist


def _tc_body(x_ref, pe_ref, cnt_ref,
             w1_ref, b1_ref, w2_ref, b2_ref,
             proj_w_ref, proj_b_ref, qg_w_ref, qg_b_ref,
             kg_w_ref, kg_b_ref, vg_w_ref, vg_b_ref,
             vqk_ref, vqv_ref,
             ff_w1_ref, ff_b1_ref, ff_w2_ref, ff_b2_ref,
             out_w_ref, out_b_ref,
             y_ref, k_scr, v_scr, bias_scr):
    i = pl.program_id(0)
    H = k_scr.shape[1]

    @pl.when(i == 0)
    def _prologue():
        k_scr[...] = vqk_ref[...] @ kg_w_ref[...] + kg_b_ref[...]
        v_scr[...] = vqv_ref[...] @ vg_w_ref[...] + vg_b_ref[...]
        cnt = jnp.sum(cnt_ref[...], axis=0).astype(jnp.float32)
        bias_scr[...] = jnp.log(cnt)[None, :]

    xb = x_ref[...]
    h = jnp.maximum(xb @ w1_ref[...] + b1_ref[...], 0.0) @ w2_ref[...] + b2_ref[...]
    p = h @ proj_w_ref[...] + proj_b_ref[...]
    GD = p.shape[1]
    q = (p @ qg_w_ref[:GD, :] + pe_ref[...] @ qg_w_ref[GD:, :]) + qg_b_ref[...]
    scale = 1.0 / math.sqrt(H)
    dots = lax.dot_general(q, k_scr[...], (((1,), (1,)), ((), ())))
    dots = dots * scale + bias_scr[...]
    m = jnp.max(dots, axis=1, keepdims=True)
    e = jnp.exp(dots - m)
    attn = e / jnp.sum(e, axis=1, keepdims=True)
    out = attn @ v_scr[...]
    f = jnp.maximum(out @ ff_w1_ref[...] + ff_b1_ref[...], 0.0)
    f = jnp.maximum(f @ ff_w2_ref[...] + ff_b2_ref[...], 0.0)
    y_ref[...] = f @ out_w_ref[...] + out_b_ref[...]


def _tc_fused(x, pos_enc, counts_parts, params, BLK=1024):
    B, IN = x.shape
    GD = pos_enc.shape[1]
    C = counts_parts.shape[1]
    H = params["qg_b"].shape[0]
    grid = (B // BLK,)

    full = lambda s: pl.BlockSpec(s, lambda i: (0,) * len(s))
    row2 = lambda d: pl.BlockSpec((BLK, d), lambda i: (i, 0))
    vec = lambda d: pl.BlockSpec((1, d), lambda i: (0, 0))

    in_specs = [
        row2(IN),
        row2(GD),
        full(counts_parts.shape),
        full((IN, H)), vec(H),
        full((H, H)), vec(H),
        full((H, GD)), vec(GD),
        full((2 * GD, H)), vec(H),
        full((2 * GD, H)), vec(H),
        full((GD, H)), vec(H),
        full((C, 2 * GD)),
        full((C, GD)),
        full((H, H)), vec(H),
        full((H, H)), vec(H),
        full((H, H)), vec(H),
    ]

    args = [
        x, pos_enc, counts_parts,
        params["fc_in_w1"], params["fc_in_b1"].reshape(1, -1),
        params["fc_in_w2"], params["fc_in_b2"].reshape(1, -1),
        params["proj_w"], params["proj_b"].reshape(1, -1),
        params["qg_w"], params["qg_b"].reshape(1, -1),
        params["kg_w"], params["kg_b"].reshape(1, -1),
        params["vg_w"], params["vg_b"].reshape(1, -1),
        params["vq_k"], params["vq_v"],
        params["ff_w1"], params["ff_b1"].reshape(1, -1),
        params["ff_w2"], params["ff_b2"].reshape(1, -1),
        params["out_w"], params["out_b"].reshape(1, -1),
    ]

    return pl.pallas_call(
        _tc_body,
        grid=grid,
        in_specs=in_specs,
        out_specs=row2(H),
        out_shape=jax.ShapeDtypeStruct((B, H), jnp.float32),
        scratch_shapes=[
            pltpu.VMEM((C, H), jnp.float32),
            pltpu.VMEM((C, H), jnp.float32),
            pltpu.VMEM((1, C), jnp.float32),
        ],
        compiler_params=pltpu.CompilerParams(
            dimension_semantics=("arbitrary",),
        ),
    )(*args)


def kernel(seq, x, pos_enc, batch_idx, c_idx,
           fc_in_w1, fc_in_b1, fc_in_w2, fc_in_b2,
           fcs_w1, fcs_b1, fcs_w2, fcs_b2,
           proj_w, proj_b, qg_w, qg_b, kg_w, kg_b, vg_w, vg_b,
           vq_k, vq_v, ff_w1, ff_b1, ff_w2, ff_b2, out_w, out_b):
    NN = c_idx.shape[0]
    C = vq_k.shape[0]
    counts_parts = _make_sc_hist(NN, C)(c_idx.astype(jnp.int32))
    params = dict(
        fc_in_w1=fc_in_w1, fc_in_b1=fc_in_b1,
        fc_in_w2=fc_in_w2, fc_in_b2=fc_in_b2,
        proj_w=proj_w, proj_b=proj_b,
        qg_w=qg_w, qg_b=qg_b, kg_w=kg_w, kg_b=kg_b,
        vg_w=vg_w, vg_b=vg_b, vq_k=vq_k, vq_v=vq_v,
        ff_w1=ff_w1, ff_b1=ff_b1, ff_w2=ff_w2, ff_b2=ff_b2,
        out_w=out_w, out_b=out_b,
    )
    return _tc_fused(x, pos_enc, counts_parts, params)


# speedup vs baseline: 1.3873x; 1.0201x over previous
import functools
import math

import jax
import jax.numpy as jnp
from jax import lax
from jax.experimental import pallas as pl
from jax.experimental.pallas import tpu as pltpu
from jax.experimental.pallas import tpu_sc as plsc


@functools.lru_cache(maxsize=None)
def _make_sc_hist(NN: int, C: int):
    L = 16
    NW = 32
    CH = (NN // NW) // L * L
    assert CH % 8 == 0
    TAIL = NN - CH * NW
    assert TAIL % L == 0 and (CH * NW) % 8 == 0

    mesh = plsc.VectorSubcoreMesh(core_axis_name="c", subcore_axis_name="s")

    @functools.partial(
        pl.kernel,
        mesh=mesh,
        out_type=jax.ShapeDtypeStruct((NW, C), jnp.int32),
        scratch_types=[
            pltpu.VMEM((CH,), jnp.int32),
            pltpu.VMEM((L * C,), jnp.int32),
            pltpu.VMEM((C,), jnp.int32),
            pltpu.VMEM((max(TAIL, L),), jnp.int32),
        ],
        compiler_params=pltpu.CompilerParams(needs_layout_passes=False),
    )
    def sc_hist(idx_hbm, out_hbm, idx_v, hist_v, part_v, tail_v):
        wid = lax.axis_index("s") * 2 + lax.axis_index("c")
        base = wid * CH
        pltpu.sync_copy(idx_hbm.at[pl.ds(base, CH)], idx_v)

        lane_base = lax.iota(jnp.int32, 16) * C
        ones = jnp.ones((L,), jnp.int32)
        zeros = jnp.zeros((L,), jnp.int32)

        def zbody(j, _):
            hist_v[pl.ds(j * L, L)] = zeros
            return _
        lax.fori_loop(0, (L * C) // L, zbody, 0, unroll=8)

        def sbody(i, _):
            vals = idx_v[pl.ds(i * L, L)]
            plsc.addupdate_scatter(hist_v, [lane_base + vals], ones)
            return _
        lax.fori_loop(0, CH // L, sbody, 0, unroll=8)

        if TAIL > 0:
            @pl.when(wid == 0)
            def _tail():
                pltpu.sync_copy(idx_hbm.at[pl.ds(CH * NW, TAIL)], tail_v)
                def tbody(i, _):
                    vals = tail_v[pl.ds(i * L, L)]
                    plsc.addupdate_scatter(hist_v, [lane_base + vals], ones)
                    return _
                lax.fori_loop(0, TAIL // L, tbody, 0, unroll=4)

        def rbody(j, _):
            acc = hist_v[pl.ds(j * L, L)]
            for l in range(1, L):
                acc = acc + hist_v[pl.ds(j * L + l * C, L)]
            part_v[pl.ds(j * L, L)] = acc
            return _
        lax.fori_loop(0, C // L, rbody, 0, unroll=4)

        pltpu.sync_copy(part_v, out_hbm.at[wid])

    return sc_hist


def _tc_body(x_ref, pe_ref, cnt_ref,
             w1_ref, b1_ref, w2_ref, b2_ref,
             proj_w_ref, proj_b_ref, qg_w_ref, qg_b_ref,
             kg_w_ref, kg_b_ref, vg_w_ref, vg_b_ref,
             vqk_ref, vqv_ref,
             ff_w1_ref, ff_b1_ref, ff_w2_ref, ff_b2_ref,
             out_w_ref, out_b_ref,
             y_ref, k_scr, v_scr, bias_scr):
    i = pl.program_id(0)
    H = k_scr.shape[1]

    @pl.when(i == 0)
    def _prologue():
        k_scr[...] = (vqk_ref[...] @ kg_w_ref[...] + kg_b_ref[...]).astype(jnp.bfloat16)
        v_scr[...] = (vqv_ref[...] @ vg_w_ref[...] + vg_b_ref[...]).astype(jnp.bfloat16)
        cnt = jnp.sum(cnt_ref[...], axis=0).astype(jnp.float32)
        bias_scr[...] = jnp.log(cnt)[None, :]

    xb = x_ref[...]
    h = jnp.maximum(xb @ w1_ref[...] + b1_ref[...], 0.0) @ w2_ref[...] + b2_ref[...]
    p = h @ proj_w_ref[...] + proj_b_ref[...]
    GD = p.shape[1]
    q = (p @ qg_w_ref[:GD, :] + pe_ref[...] @ qg_w_ref[GD:, :]) + qg_b_ref[...]
    scale = 1.0 / math.sqrt(H)
    dots = lax.dot_general(q.astype(jnp.bfloat16), k_scr[...],
                           (((1,), (1,)), ((), ())),
                           preferred_element_type=jnp.float32)
    dots = dots * scale + bias_scr[...]
    m = jnp.max(dots, axis=1, keepdims=True)
    e = jnp.exp(dots - m)
    num = lax.dot_general(e.astype(jnp.bfloat16), v_scr[...],
                          (((1,), (0,)), ((), ())),
                          preferred_element_type=jnp.float32)
    out = num / jnp.sum(e, axis=1, keepdims=True)
    f = jnp.maximum(out @ ff_w1_ref[...] + ff_b1_ref[...], 0.0)
    f = jnp.maximum(f @ ff_w2_ref[...] + ff_b2_ref[...], 0.0)
    y_ref[...] = f @ out_w_ref[...] + out_b_ref[...]


def _tc_fused(x, pos_enc, counts_parts, params, BLK=1024):
    B, IN = x.shape
    GD = pos_enc.shape[1]
    C = counts_parts.shape[1]
    H = params["qg_b"].shape[0]
    grid = (B // BLK,)

    full = lambda s: pl.BlockSpec(s, lambda i: (0,) * len(s))
    row2 = lambda d: pl.BlockSpec((BLK, d), lambda i: (i, 0))
    vec = lambda d: pl.BlockSpec((1, d), lambda i: (0, 0))

    in_specs = [
        row2(IN),
        row2(GD),
        full(counts_parts.shape),
        full((IN, H)), vec(H),
        full((H, H)), vec(H),
        full((H, GD)), vec(GD),
        full((2 * GD, H)), vec(H),
        full((2 * GD, H)), vec(H),
        full((GD, H)), vec(H),
        full((C, 2 * GD)),
        full((C, GD)),
        full((H, H)), vec(H),
        full((H, H)), vec(H),
        full((H, H)), vec(H),
    ]

    args = [
        x, pos_enc, counts_parts,
        params["fc_in_w1"], params["fc_in_b1"].reshape(1, -1),
        params["fc_in_w2"], params["fc_in_b2"].reshape(1, -1),
        params["proj_w"], params["proj_b"].reshape(1, -1),
        params["qg_w"], params["qg_b"].reshape(1, -1),
        params["kg_w"], params["kg_b"].reshape(1, -1),
        params["vg_w"], params["vg_b"].reshape(1, -1),
        params["vq_k"], params["vq_v"],
        params["ff_w1"], params["ff_b1"].reshape(1, -1),
        params["ff_w2"], params["ff_b2"].reshape(1, -1),
        params["out_w"], params["out_b"].reshape(1, -1),
    ]

    return pl.pallas_call(
        _tc_body,
        grid=grid,
        in_specs=in_specs,
        out_specs=row2(H),
        out_shape=jax.ShapeDtypeStruct((B, H), jnp.float32),
        scratch_shapes=[
            pltpu.VMEM((C, H), jnp.bfloat16),
            pltpu.VMEM((C, H), jnp.bfloat16),
            pltpu.VMEM((1, C), jnp.float32),
        ],
        compiler_params=pltpu.CompilerParams(
            dimension_semantics=("arbitrary",),
        ),
    )(*args)


def kernel(seq, x, pos_enc, batch_idx, c_idx,
           fc_in_w1, fc_in_b1, fc_in_w2, fc_in_b2,
           fcs_w1, fcs_b1, fcs_w2, fcs_b2,
           proj_w, proj_b, qg_w, qg_b, kg_w, kg_b, vg_w, vg_b,
           vq_k, vq_v, ff_w1, ff_b1, ff_w2, ff_b2, out_w, out_b):
    NN = c_idx.shape[0]
    C = vq_k.shape[0]
    counts_parts = _make_sc_hist(NN, C)(c_idx.astype(jnp.int32))
    params = dict(
        fc_in_w1=fc_in_w1, fc_in_b1=fc_in_b1,
        fc_in_w2=fc_in_w2, fc_in_b2=fc_in_b2,
        proj_w=proj_w, proj_b=proj_b,
        qg_w=qg_w, qg_b=qg_b, kg_w=kg_w, kg_b=kg_b,
        vg_w=vg_w, vg_b=vg_b, vq_k=vq_k, vq_v=vq_v,
        ff_w1=ff_w1, ff_b1=ff_b1, ff_w2=ff_w2, ff_b2=ff_b2,
        out_w=out_w, out_b=out_b,
    )
    return _tc_fused(x, pos_enc, counts_parts, params)
